# baseline (speedup 1.0000x reference)
import functools

import jax
import jax.numpy as jnp
from jax import lax
from jax.experimental import pallas as pl
from jax.experimental.pallas import tpu as pltpu
from jax.experimental.pallas import tpu_sc as plsc

N_NODES = 10000
N_EDGES = 320000
HID = 128
NC = 2
NS = 16
NPAD = 10240
ROWS_PER_TILE = NPAD // NS
K1, NBUF1 = 40, 5
K2, NBUF2 = 40, 5
KD = 80
NDEG = 10

_f32 = jnp.float32
_mesh = plsc.VectorSubcoreMesh(core_axis_name="c", subcore_axis_name="s")


def _zero_rows(zbuf, nrows, width):
    zeros16 = jnp.zeros((16,), _f32)

    def body(r, _):
        for w in range(width // 16):
            zbuf[r, pl.ds(w * 16, 16)] = zeros16
        return 0

    lax.fori_loop(0, nrows, body, 0)


@functools.partial(
    pl.kernel,
    out_type=jax.ShapeDtypeStruct((NC * NPAD,), _f32),
    mesh=_mesh,
    scratch_types=(
        [pltpu.VMEM_SHARED((NPAD,), _f32)]
        + [pltpu.VMEM((KD,), jnp.int32) for _ in range(NDEG)]
        + [
            pltpu.VMEM((KD,), _f32),
            pltpu.VMEM((ROWS_PER_TILE,), _f32),
            pltpu.SemaphoreType.DMA,
            pltpu.SemaphoreType.DMA,
        ]
    ),
)
def _deg_kernel(edge_hbm, out_hbm, acc, *rest):
    idx_v = rest[:NDEG]
    ones_v, zrow_v, isem, ssem = rest[NDEG:]
    c = lax.axis_index("c")
    s = lax.axis_index("s")
    ones16 = jnp.ones((16,), _f32)
    zeros16 = jnp.zeros((16,), _f32)
    for w in range(KD // 16):
        ones_v[pl.ds(w * 16, 16)] = ones16

    def zbody(w, _):
        zrow_v[pl.ds(w * 16, 16)] = zeros16
        return 0

    lax.fori_loop(0, ROWS_PER_TILE // 16, zbody, 0)
    pltpu.sync_copy(zrow_v, acc.at[pl.ds(s * ROWS_PER_TILE, ROWS_PER_TILE)])
    plsc.subcore_barrier()

    per_tile = N_EDGES // NS

    def drain_scatters():
        for b in range(NDEG):
            pltpu.make_async_copy(edge_hbm.at[pl.ds(0, KD)], idx_v[b],
                                  ssem).wait()

    def group_body(g, _):
        gbase = c * N_EDGES + s * per_tile + g * (NDEG * KD)

        @pl.when(g > 0)
        def _():
            drain_scatters()

        loads = [pltpu.async_copy(edge_hbm.at[pl.ds(gbase + b * KD, KD)],
                                  idx_v[b], isem) for b in range(NDEG)]
        for cp in loads:
            cp.wait()
        for b in range(NDEG):
            pltpu.async_copy(ones_v, acc.at[idx_v[b]], ssem, add=True)
        return 0

    lax.fori_loop(0, per_tile // (NDEG * KD), group_body, 0)
    drain_scatters()
    plsc.subcore_barrier()
    pltpu.sync_copy(acc.at[pl.ds(s * ROWS_PER_TILE, ROWS_PER_TILE)],
                    out_hbm.at[pl.ds(c * NPAD + s * ROWS_PER_TILE, ROWS_PER_TILE)])


def _agg_body(edge_hbm, h_hbm, out_hbm, acc, *rest, feature_split, k, nbuf):
    idxs_v = [rest[:nbuf], rest[nbuf:2 * nbuf]]
    idxd_v = [rest[2 * nbuf:3 * nbuf], rest[3 * nbuf:4 * nbuf]]
    rows_v = rest[4 * nbuf]
    isem = rest[4 * nbuf + 1:4 * nbuf + 3]
    gsem = rest[4 * nbuf + 3:4 * nbuf + 3 + nbuf]
    ssem = rest[4 * nbuf + 3 + nbuf:]
    group = nbuf * k
    c = lax.axis_index("c")
    s = lax.axis_index("s")
    _zero_rows(rows_v.at[0], k, HID)
    for t in range(ROWS_PER_TILE // k):
        pltpu.sync_copy(rows_v.at[0], acc.at[pl.ds(s * ROWS_PER_TILE + t * k, k)])
    plsc.subcore_barrier()

    if feature_split:
        per_tile = N_EDGES // NS
        e0 = s * per_tile
        table = h_hbm.at[c]
    else:
        per_tile = N_EDGES // (NS * NC)
        e0 = (s * NC + c) * per_tile
        table = h_hbm
    ngroups = per_tile // group
    assert per_tile % group == 0 and ngroups % 2 == 0 and ROWS_PER_TILE % k == 0

    def fire_idx(gbase, slot):
        for b in range(nbuf):
            pltpu.async_copy(edge_hbm.at[pl.ds(gbase + b * k, k)],
                             idxs_v[slot][b], isem[slot])
            pltpu.async_copy(edge_hbm.at[pl.ds(N_EDGES + gbase + b * k, k)],
                             idxd_v[slot][b], isem[slot])

    def wait_idx(slot):
        for b in range(nbuf):
            pltpu.make_async_copy(edge_hbm.at[pl.ds(0, k)], idxs_v[slot][b],
                                  isem[slot]).wait()
            pltpu.make_async_copy(edge_hbm.at[pl.ds(0, k)], idxd_v[slot][b],
                                  isem[slot]).wait()

    def drain_scatters():
        for b in range(nbuf):
            pltpu.make_async_copy(table.at[pl.ds(0, k)], rows_v.at[b],
                                  ssem[b]).wait()

    fire_idx(e0, 0)

    def pair_body(t, _):
        for u in (0, 1):
            g = 2 * t + u

            if u == 0:
                @pl.when(t > 0)
                def _():
                    drain_scatters()
            else:
                drain_scatters()

            wait_idx(u)

            @pl.when(g + 1 < ngroups)
            def _():
                fire_idx(e0 + (g + 1) * group, 1 - u)

            gathers = [pltpu.async_copy(table.at[idxs_v[u][b]], rows_v.at[b],
                                        gsem[b]) for b in range(nbuf)]
            for b in range(nbuf):
                gathers[b].wait()
                pltpu.async_copy(rows_v.at[b], acc.at[idxd_v[u][b]], ssem[b],
                                 add=True)
        return 0

    lax.fori_loop(0, ngroups // 2, pair_body, 0)
    drain_scatters()
    plsc.subcore_barrier()
    pltpu.sync_copy(acc.at[pl.ds(s * ROWS_PER_TILE, ROWS_PER_TILE)],
                    out_hbm.at[c, pl.ds(s * ROWS_PER_TILE, ROWS_PER_TILE)])


def _make_agg(feature_split, k, nbuf):
    scratch = (
        [pltpu.VMEM_SHARED((NPAD, HID), _f32)]
        + [pltpu.VMEM((k,), jnp.int32) for _ in range(4 * nbuf)]
        + [pltpu.VMEM((nbuf, k, HID), _f32)]
        + [pltpu.SemaphoreType.DMA for _ in range(2 + 2 * nbuf)]
    )
    return pl.kernel(
        functools.partial(_agg_body, feature_split=feature_split, k=k, nbuf=nbuf),
        out_type=jax.ShapeDtypeStruct((NC, NPAD, HID), _f32),
        mesh=_mesh,
        scratch_types=scratch,
    )


_agg1_kernel = _make_agg(feature_split=True, k=K1, nbuf=NBUF1)
_agg2_kernel = _make_agg(feature_split=False, k=K2, nbuf=NBUF2)


R = 1000
GRID = N_NODES // R


def _norm(deg):
    return jnp.where(deg > 0, lax.rsqrt(jnp.maximum(deg, 1.0)), 0.0)


def _tc1a_body(x_ref, st_ref, w_ref, wc1_ref, out_ref, pre2_ref):
    x = jnp.concatenate([x_ref[...], st_ref[...]], axis=1)
    out_ref[0] = jnp.dot(x, w_ref[...], preferred_element_type=_f32)
    pre2_ref[...] = jnp.dot(x_ref[...], wc1_ref[...], preferred_element_type=_f32)


def _tc1a(inputs, states, W_ru, W_c1):
    return pl.pallas_call(
        _tc1a_body,
        grid=(GRID, NC),
        in_specs=[
            pl.BlockSpec((R, HID), lambda i, h: (i, 0)),
            pl.BlockSpec((R, HID), lambda i, h: (i, 0)),
            pl.BlockSpec((2 * HID, HID), lambda i, h: (0, h)),
            pl.BlockSpec((HID, HID), lambda i, h: (0, 0)),
        ],
        out_specs=[
            pl.BlockSpec((1, R, HID), lambda i, h: (h, i, 0)),
            pl.BlockSpec((R, HID), lambda i, h: (i, 0)),
        ],
        out_shape=[
            jax.ShapeDtypeStruct((NC, N_NODES, HID), _f32),
            jax.ShapeDtypeStruct((N_NODES, HID), _f32),
        ],
    )(inputs, states, W_ru, W_c1)


def _tc1b_body(hr_ref, dgo_ref, out_ref):
    out_ref[...] = hr_ref[...] * _norm(dgo_ref[...])


def _tc1b(h_raw, deg_out):
    return pl.pallas_call(
        _tc1b_body,
        grid=(GRID, NC),
        in_specs=[
            pl.BlockSpec((1, R, HID), lambda i, h: (h, i, 0)),
            pl.BlockSpec((R, 1), lambda i, h: (i, 0)),
        ],
        out_specs=pl.BlockSpec((1, R, HID), lambda i, h: (h, i, 0)),
        out_shape=jax.ShapeDtypeStruct((NC, N_NODES, HID), _f32),
    )(h_raw, deg_out)


def _tc2_body(agg_ref, dgi_ref, dgo_ref, bru_ref, pre2_ref, st_ref, wc2_ref,
              u_ref, h2_ref):
    ndst = _norm(dgi_ref[...])
    nsrc = _norm(dgo_ref[...])
    r = jax.nn.sigmoid(agg_ref[0] * ndst + bru_ref[0, :HID])
    u = jax.nn.sigmoid(agg_ref[1] * ndst + bru_ref[0, HID:])
    u_ref[...] = u
    h2 = pre2_ref[...] + jnp.dot(r * st_ref[...], wc2_ref[...],
                                 preferred_element_type=_f32)
    h2_ref[...] = h2 * nsrc


def _tc2(agg1, deg_in, deg_out, b_ru, pre2, states, W_c2):
    return pl.pallas_call(
        _tc2_body,
        grid=(GRID,),
        in_specs=[
            pl.BlockSpec((NC, R, HID), lambda i: (0, i, 0)),
            pl.BlockSpec((R, 1), lambda i: (i, 0)),
            pl.BlockSpec((R, 1), lambda i: (i, 0)),
            pl.BlockSpec((1, 2 * HID), lambda i: (0, 0)),
            pl.BlockSpec((R, HID), lambda i: (i, 0)),
            pl.BlockSpec((R, HID), lambda i: (i, 0)),
            pl.BlockSpec((HID, HID), lambda i: (0, 0)),
        ],
        out_specs=[
            pl.BlockSpec((R, HID), lambda i: (i, 0)),
            pl.BlockSpec((R, HID), lambda i: (i, 0)),
        ],
        out_shape=[
            jax.ShapeDtypeStruct((N_NODES, HID), _f32),
            jax.ShapeDtypeStruct((N_NODES, HID), _f32),
        ],
    )(agg1, deg_in, deg_out, b_ru, pre2, states, W_c2)


def _tc3_body(agg_ref, dgi_ref, bc_ref, u_ref, st_ref, out_ref):
    ndst = _norm(dgi_ref[...])
    agg = agg_ref[0] + agg_ref[1]
    cand = jnp.tanh(agg * ndst + bc_ref[0])
    u = u_ref[...]
    out_ref[...] = u * st_ref[...] + (1.0 - u) * cand


def _tc3(agg2, deg_in, b_c, u, states):
    return pl.pallas_call(
        _tc3_body,
        grid=(GRID,),
        in_specs=[
            pl.BlockSpec((NC, R, HID), lambda i: (0, i, 0)),
            pl.BlockSpec((R, 1), lambda i: (i, 0)),
            pl.BlockSpec((1, HID), lambda i: (0, 0)),
            pl.BlockSpec((R, HID), lambda i: (i, 0)),
            pl.BlockSpec((R, HID), lambda i: (i, 0)),
        ],
        out_specs=pl.BlockSpec((R, HID), lambda i: (i, 0)),
        out_shape=jax.ShapeDtypeStruct((N_NODES, HID), _f32),
    )(agg2, deg_in, b_c, u, states)


def kernel(inputs, states, edge_index, W_ru, b_ru, W_c, b_c):
    edge_flat = edge_index.reshape(2 * N_EDGES)
    deg = _deg_kernel(edge_flat)
    deg_out = deg[:N_NODES, None]
    deg_in = deg[NPAD:NPAD + N_NODES, None]
    h_raw, pre2 = _tc1a(inputs, states, W_ru, W_c[:HID])
    h1 = _tc1b(h_raw, deg_out)
    agg1 = _agg1_kernel(edge_flat, h1)
    u, h2 = _tc2(agg1, deg_in, deg_out, b_ru[None, :], pre2, states, W_c[HID:])
    agg2 = _agg2_kernel(edge_flat, h2)
    out = _tc3(agg2, deg_in, b_c[None, :], u, states)
    return (out, out)


# speedup vs baseline: 1.0016x; 1.0016x over previous
import functools

import jax
import jax.numpy as jnp
from jax import lax
from jax.experimental import pallas as pl
from jax.experimental.pallas import tpu as pltpu
from jax.experimental.pallas import tpu_sc as plsc

N_NODES = 10000
N_EDGES = 320000
HID = 128
NC = 2
NS = 16
NPAD = 10240
ROWS_PER_TILE = NPAD // NS
K1, NBUF1 = 40, 5
K2, NBUF2 = 40, 5
KD = 80
NDEG = 10

_f32 = jnp.float32
_mesh = plsc.VectorSubcoreMesh(core_axis_name="c", subcore_axis_name="s")


def _zero_rows(zbuf, nrows, width):
    zeros16 = jnp.zeros((16,), _f32)

    def body(r, _):
        for w in range(width // 16):
            zbuf[r, pl.ds(w * 16, 16)] = zeros16
        return 0

    lax.fori_loop(0, nrows, body, 0)


@functools.partial(
    pl.kernel,
    out_type=jax.ShapeDtypeStruct((NC * NPAD,), _f32),
    mesh=_mesh,
    scratch_types=(
        [pltpu.VMEM_SHARED((NPAD,), _f32)]
        + [pltpu.VMEM((KD,), jnp.int32) for _ in range(NDEG)]
        + [
            pltpu.VMEM((KD,), _f32),
            pltpu.VMEM((ROWS_PER_TILE,), _f32),
            pltpu.SemaphoreType.DMA,
            pltpu.SemaphoreType.DMA,
        ]
    ),
)
def _deg_kernel(edge_hbm, out_hbm, acc, *rest):
    idx_v = rest[:NDEG]
    ones_v, zrow_v, isem, ssem = rest[NDEG:]
    c = lax.axis_index("c")
    s = lax.axis_index("s")
    ones16 = jnp.ones((16,), _f32)
    zeros16 = jnp.zeros((16,), _f32)
    for w in range(KD // 16):
        ones_v[pl.ds(w * 16, 16)] = ones16

    def zbody(w, _):
        zrow_v[pl.ds(w * 16, 16)] = zeros16
        return 0

    lax.fori_loop(0, ROWS_PER_TILE // 16, zbody, 0)
    pltpu.sync_copy(zrow_v, acc.at[pl.ds(s * ROWS_PER_TILE, ROWS_PER_TILE)])
    plsc.subcore_barrier()

    per_tile = N_EDGES // NS

    def drain_scatters():
        for b in range(NDEG):
            pltpu.make_async_copy(edge_hbm.at[pl.ds(0, KD)], idx_v[b],
                                  ssem).wait()

    def group_body(g, _):
        gbase = c * N_EDGES + s * per_tile + g * (NDEG * KD)

        @pl.when(g > 0)
        def _():
            drain_scatters()

        loads = [pltpu.async_copy(edge_hbm.at[pl.ds(gbase + b * KD, KD)],
                                  idx_v[b], isem) for b in range(NDEG)]
        for cp in loads:
            cp.wait()
        for b in range(NDEG):
            pltpu.async_copy(ones_v, acc.at[idx_v[b]], ssem, add=True)
        return 0

    lax.fori_loop(0, per_tile // (NDEG * KD), group_body, 0)
    drain_scatters()
    plsc.subcore_barrier()
    pltpu.sync_copy(acc.at[pl.ds(s * ROWS_PER_TILE, ROWS_PER_TILE)],
                    out_hbm.at[pl.ds(c * NPAD + s * ROWS_PER_TILE, ROWS_PER_TILE)])


def _agg_body(edge_hbm, h_hbm, out_hbm, acc, *rest, feature_split, k, nbuf):
    idxs_v = [rest[:nbuf], rest[nbuf:2 * nbuf]]
    idxd_v = [rest[2 * nbuf:3 * nbuf], rest[3 * nbuf:4 * nbuf]]
    rows_v = rest[4 * nbuf]
    isem = rest[4 * nbuf + 1:4 * nbuf + 3]
    gsem = rest[4 * nbuf + 3:4 * nbuf + 3 + nbuf]
    ssem = rest[4 * nbuf + 3 + nbuf:]
    group = nbuf * k
    c = lax.axis_index("c")
    s = lax.axis_index("s")
    _zero_rows(rows_v.at[0], k, HID)
    zcopies = [
        pltpu.async_copy(rows_v.at[0],
                         acc.at[pl.ds(s * ROWS_PER_TILE + t * k, k)], isem[0])
        for t in range(ROWS_PER_TILE // k)
    ]
    for cp in zcopies:
        cp.wait()
    plsc.subcore_barrier()

    if feature_split:
        per_tile = N_EDGES // NS
        e0 = s * per_tile
        table = h_hbm.at[c]
    else:
        per_tile = N_EDGES // (NS * NC)
        e0 = (s * NC + c) * per_tile
        table = h_hbm
    ngroups = per_tile // group
    assert per_tile % group == 0 and ngroups % 2 == 0 and ROWS_PER_TILE % k == 0

    def fire_idx(gbase, slot):
        for b in range(nbuf):
            pltpu.async_copy(edge_hbm.at[pl.ds(gbase + b * k, k)],
                             idxs_v[slot][b], isem[slot])
            pltpu.async_copy(edge_hbm.at[pl.ds(N_EDGES + gbase + b * k, k)],
                             idxd_v[slot][b], isem[slot])

    def wait_idx(slot):
        for b in range(nbuf):
            pltpu.make_async_copy(edge_hbm.at[pl.ds(0, k)], idxs_v[slot][b],
                                  isem[slot]).wait()
            pltpu.make_async_copy(edge_hbm.at[pl.ds(0, k)], idxd_v[slot][b],
                                  isem[slot]).wait()

    def drain_scatters():
        for b in range(nbuf):
            pltpu.make_async_copy(table.at[pl.ds(0, k)], rows_v.at[b],
                                  ssem[b]).wait()

    fire_idx(e0, 0)

    def pair_body(t, _):
        for u in (0, 1):
            g = 2 * t + u

            if u == 0:
                @pl.when(t > 0)
                def _():
                    drain_scatters()
            else:
                drain_scatters()

            wait_idx(u)

            @pl.when(g + 1 < ngroups)
            def _():
                fire_idx(e0 + (g + 1) * group, 1 - u)

            gathers = [pltpu.async_copy(table.at[idxs_v[u][b]], rows_v.at[b],
                                        gsem[b]) for b in range(nbuf)]
            for b in range(nbuf):
                gathers[b].wait()
                pltpu.async_copy(rows_v.at[b], acc.at[idxd_v[u][b]], ssem[b],
                                 add=True)
        return 0

    lax.fori_loop(0, ngroups // 2, pair_body, 0)
    drain_scatters()
    plsc.subcore_barrier()
    pltpu.sync_copy(acc.at[pl.ds(s * ROWS_PER_TILE, ROWS_PER_TILE)],
                    out_hbm.at[c, pl.ds(s * ROWS_PER_TILE, ROWS_PER_TILE)])


def _make_agg(feature_split, k, nbuf):
    scratch = (
        [pltpu.VMEM_SHARED((NPAD, HID), _f32)]
        + [pltpu.VMEM((k,), jnp.int32) for _ in range(4 * nbuf)]
        + [pltpu.VMEM((nbuf, k, HID), _f32)]
        + [pltpu.SemaphoreType.DMA for _ in range(2 + 2 * nbuf)]
    )
    return pl.kernel(
        functools.partial(_agg_body, feature_split=feature_split, k=k, nbuf=nbuf),
        out_type=jax.ShapeDtypeStruct((NC, NPAD, HID), _f32),
        mesh=_mesh,
        scratch_types=scratch,
    )


_agg1_kernel = _make_agg(feature_split=True, k=K1, nbuf=NBUF1)
_agg2_kernel = _make_agg(feature_split=False, k=K2, nbuf=NBUF2)


R = 1000
GRID = N_NODES // R


def _norm(deg):
    return jnp.where(deg > 0, lax.rsqrt(jnp.maximum(deg, 1.0)), 0.0)


def _tc1a_body(x_ref, st_ref, w_ref, wc1_ref, out_ref, pre2_ref):
    x = jnp.concatenate([x_ref[...], st_ref[...]], axis=1)
    out_ref[0] = jnp.dot(x, w_ref[...], preferred_element_type=_f32)
    pre2_ref[...] = jnp.dot(x_ref[...], wc1_ref[...], preferred_element_type=_f32)


def _tc1a(inputs, states, W_ru, W_c1):
    return pl.pallas_call(
        _tc1a_body,
        grid=(GRID, NC),
        in_specs=[
            pl.BlockSpec((R, HID), lambda i, h: (i, 0)),
            pl.BlockSpec((R, HID), lambda i, h: (i, 0)),
            pl.BlockSpec((2 * HID, HID), lambda i, h: (0, h)),
            pl.BlockSpec((HID, HID), lambda i, h: (0, 0)),
        ],
        out_specs=[
            pl.BlockSpec((1, R, HID), lambda i, h: (h, i, 0)),
            pl.BlockSpec((R, HID), lambda i, h: (i, 0)),
        ],
        out_shape=[
            jax.ShapeDtypeStruct((NC, N_NODES, HID), _f32),
            jax.ShapeDtypeStruct((N_NODES, HID), _f32),
        ],
    )(inputs, states, W_ru, W_c1)


def _tc1b_body(hr_ref, dgo_ref, out_ref):
    out_ref[...] = hr_ref[...] * _norm(dgo_ref[...])


def _tc1b(h_raw, deg_out):
    return pl.pallas_call(
        _tc1b_body,
        grid=(GRID, NC),
        in_specs=[
            pl.BlockSpec((1, R, HID), lambda i, h: (h, i, 0)),
            pl.BlockSpec((R, 1), lambda i, h: (i, 0)),
        ],
        out_specs=pl.BlockSpec((1, R, HID), lambda i, h: (h, i, 0)),
        out_shape=jax.ShapeDtypeStruct((NC, N_NODES, HID), _f32),
    )(h_raw, deg_out)


def _tc2_body(agg_ref, dgi_ref, dgo_ref, bru_ref, pre2_ref, st_ref, wc2_ref,
              u_ref, h2_ref):
    ndst = _norm(dgi_ref[...])
    nsrc = _norm(dgo_ref[...])
    r = jax.nn.sigmoid(agg_ref[0] * ndst + bru_ref[0, :HID])
    u = jax.nn.sigmoid(agg_ref[1] * ndst + bru_ref[0, HID:])
    u_ref[...] = u
    h2 = pre2_ref[...] + jnp.dot(r * st_ref[...], wc2_ref[...],
                                 preferred_element_type=_f32)
    h2_ref[...] = h2 * nsrc


def _tc2(agg1, deg_in, deg_out, b_ru, pre2, states, W_c2):
    return pl.pallas_call(
        _tc2_body,
        grid=(GRID,),
        in_specs=[
            pl.BlockSpec((NC, R, HID), lambda i: (0, i, 0)),
            pl.BlockSpec((R, 1), lambda i: (i, 0)),
            pl.BlockSpec((R, 1), lambda i: (i, 0)),
            pl.BlockSpec((1, 2 * HID), lambda i: (0, 0)),
            pl.BlockSpec((R, HID), lambda i: (i, 0)),
            pl.BlockSpec((R, HID), lambda i: (i, 0)),
            pl.BlockSpec((HID, HID), lambda i: (0, 0)),
        ],
        out_specs=[
            pl.BlockSpec((R, HID), lambda i: (i, 0)),
            pl.BlockSpec((R, HID), lambda i: (i, 0)),
        ],
        out_shape=[
            jax.ShapeDtypeStruct((N_NODES, HID), _f32),
            jax.ShapeDtypeStruct((N_NODES, HID), _f32),
        ],
    )(agg1, deg_in, deg_out, b_ru, pre2, states, W_c2)


def _tc3_body(agg_ref, dgi_ref, bc_ref, u_ref, st_ref, out_ref):
    ndst = _norm(dgi_ref[...])
    agg = agg_ref[0] + agg_ref[1]
    cand = jnp.tanh(agg * ndst + bc_ref[0])
    u = u_ref[...]
    out_ref[...] = u * st_ref[...] + (1.0 - u) * cand


def _tc3(agg2, deg_in, b_c, u, states):
    return pl.pallas_call(
        _tc3_body,
        grid=(GRID,),
        in_specs=[
            pl.BlockSpec((NC, R, HID), lambda i: (0, i, 0)),
            pl.BlockSpec((R, 1), lambda i: (i, 0)),
            pl.BlockSpec((1, HID), lambda i: (0, 0)),
            pl.BlockSpec((R, HID), lambda i: (i, 0)),
            pl.BlockSpec((R, HID), lambda i: (i, 0)),
        ],
        out_specs=pl.BlockSpec((R, HID), lambda i: (i, 0)),
        out_shape=jax.ShapeDtypeStruct((N_NODES, HID), _f32),
    )(agg2, deg_in, b_c, u, states)


def kernel(inputs, states, edge_index, W_ru, b_ru, W_c, b_c):
    edge_flat = edge_index.reshape(2 * N_EDGES)
    deg = _deg_kernel(edge_flat)
    deg_out = deg[:N_NODES, None]
    deg_in = deg[NPAD:NPAD + N_NODES, None]
    h_raw, pre2 = _tc1a(inputs, states, W_ru, W_c[:HID])
    h1 = _tc1b(h_raw, deg_out)
    agg1 = _agg1_kernel(edge_flat, h1)
    u, h2 = _tc2(agg1, deg_in, deg_out, b_ru[None, :], pre2, states, W_c[HID:])
    agg2 = _agg2_kernel(edge_flat, h2)
    out = _tc3(agg2, deg_in, b_c[None, :], u, states)
    return (out, out)
